# baseline (speedup 1.0000x reference)
import jax
import jax.numpy as jnp
from jax import lax
from jax.experimental import pallas as pl
from jax.experimental.pallas import tpu as pltpu
from jax.experimental.pallas import tpu_sc as plsc

FEAT_H = 16
FEAT_W = 16
HW = FEAT_H * FEAT_W
GAUSS_TH = 0.6
EXPAND = 1.0
SIGMA = 1.6
BIN_OFF = 0.5
RADIUS = 1.0
NEG_LOG_TH = 0.5108255840295616
TWO_SIG2 = 2.0 * SIGMA ** 2
IMGS = 2
N_PER_IMG = 512
G_PER_IMG = 64

_RPT = 64


def _take(v, idx):
    return lax.gather(
        v, idx[:, None],
        lax.GatherDimensionNumbers(offset_dims=(), collapsed_slice_dims=(0,),
                                   start_index_map=(0,)),
        (1,), mode=lax.GatherScatterMode.PROMISE_IN_BOUNDS)


def _gather_sc(gid_h, kx_h, ky_h, kv_h,
               kxg_h, kyg_h, kvg_h,
               gid_v, kx_v, ky_v, kv_v, kxg_v, kyg_v, kvg_v):
    wid = lax.axis_index("s") * 2 + lax.axis_index("c")
    base = wid * _RPT
    img = wid // (N_PER_IMG // _RPT)
    pltpu.sync_copy(gid_h.at[pl.ds(base, _RPT)], gid_v)
    pltpu.sync_copy(kx_h.at[pl.ds(img * G_PER_IMG, G_PER_IMG)], kx_v)
    pltpu.sync_copy(ky_h.at[pl.ds(img * G_PER_IMG, G_PER_IMG)], ky_v)
    pltpu.sync_copy(kv_h.at[pl.ds(img * G_PER_IMG, G_PER_IMG)], kv_v)

    zero16 = jnp.zeros((16,), jnp.float32)
    kxc = [kx_v[pl.ds(c * 16, 16)] for c in range(4)]
    kyc = [ky_v[pl.ds(c * 16, 16)] for c in range(4)]
    kvc = [kv_v[pl.ds(c * 16, 16)] for c in range(4)]

    def tab_gather(chunks, lo, hi):
        acc = zero16
        for c in range(4):
            acc = jnp.where(hi == c, _take(chunks[c], lo), acc)
        return acc

    for g in range(_RPT // 16):
        sl = pl.ds(g * 16, 16)
        gid16 = gid_v[sl]
        lo = gid16 & 15
        hi = lax.shift_right_logical(gid16, 4)
        kxg_v[sl] = tab_gather(kxc, lo, hi)
        kyg_v[sl] = tab_gather(kyc, lo, hi)
        kvg_v[sl] = tab_gather(kvc, lo, hi)

    pltpu.sync_copy(kxg_v, kxg_h.at[pl.ds(base, _RPT)])
    pltpu.sync_copy(kyg_v, kyg_h.at[pl.ds(base, _RPT)])
    pltpu.sync_copy(kvg_v, kvg_h.at[pl.ds(base, _RPT)])


def _label_kernel(boxes_ref, kx_ref, ky_ref, kv_ref, flag_ref,
                  cls_ref, clsw_ref, reg_ref, regw_ref):
    rows = IMGS * N_PER_IMG
    boxes = boxes_ref[...].reshape(rows, 4)
    flag = flag_ref[...].reshape(rows, 1)
    kx = kx_ref[...].reshape(rows, 1)
    ky = ky_ref[...].reshape(rows, 1)
    kv = kv_ref[...].reshape(rows, 1)

    x1 = boxes[:, 0:1]
    y1 = boxes[:, 1:2]
    x2 = boxes[:, 2:3]
    y2 = boxes[:, 3:4]
    cx = (x1 + x2) * 0.5
    cy = (y1 + y2) * 0.5
    w = (x2 - x1 + 1.0) * EXPAND
    h = (y2 - y1 + 1.0) * EXPAND
    bx1 = cx - (w - 1.0) * 0.5
    by1 = cy - (h - 1.0) * 0.5
    bx2 = cx + (w - 1.0) * 0.5
    by2 = cy + (h - 1.0) * 0.5

    sx = FEAT_W / (bx2 - bx1 + 1.0)
    sy = FEAT_H / (by2 - by1 + 1.0)
    x0 = (kx - bx1) * sx
    y0 = (ky - by1) * sy

    col = lax.broadcasted_iota(jnp.int32, (rows, HW), 1)
    bin_x = (col % FEAT_W).astype(jnp.float32)
    bin_y = (col // FEAT_W).astype(jnp.float32)

    dx = bin_x + BIN_OFF - x0
    dy = bin_y + BIN_OFF - y0
    inv2s2 = 1.0 / TWO_SIG2
    arg = dx * dx * inv2s2 + dy * dy * inv2s2
    score = jnp.exp(-arg)
    keep = arg <= NEG_LOG_TH

    vis = kv != 0.0
    pos = flag > 0
    active = pos & vis & jnp.any(keep, axis=-1, keepdims=True)

    cls_ref[...] = jnp.where(active, score, -1.0).reshape(IMGS, N_PER_IMG, HW)
    clsw_ref[...] = (jnp.where(active, 1.0, 0.0) * jnp.ones_like(score)
                     ).reshape(IMGS, N_PER_IMG, HW)

    m = pos & vis & keep
    off_x = (x0 - bin_x) / RADIUS
    off_y = (y0 - bin_y) / RADIUS
    zeros = jnp.zeros_like(score)
    reg = jnp.concatenate([jnp.where(m, off_x, zeros),
                           jnp.where(m, off_y, zeros)], axis=1)
    reg_ref[...] = reg.reshape(IMGS, N_PER_IMG, 2 * HW)
    rw = jnp.where(m, 1.0, 0.0)
    regw_ref[...] = jnp.concatenate([rw, rw], axis=1
                                    ).reshape(IMGS, N_PER_IMG, 2 * HW)


def kernel(boxes, gt_boxes, match_pos_flag, match_gt_id):
    B, N = boxes.shape[:2]
    KPS = 1
    BN = B * N

    flag = match_pos_flag.astype(jnp.int32).reshape(B, N, 1)
    gid1 = match_gt_id.astype(jnp.int32).reshape(BN)
    gt_f = gt_boxes.reshape(B * G_PER_IMG, 8)

    mesh = plsc.VectorSubcoreMesh(core_axis_name="c", subcore_axis_name="s",
                                  num_cores=2)
    kxg, kyg, kvg = pl.kernel(
        _gather_sc,
        out_type=(
            jax.ShapeDtypeStruct((BN,), jnp.float32),
            jax.ShapeDtypeStruct((BN,), jnp.float32),
            jax.ShapeDtypeStruct((BN,), jnp.float32),
        ),
        mesh=mesh,
        scratch_types=[
            pltpu.VMEM((_RPT,), jnp.int32),
            pltpu.VMEM((G_PER_IMG,), jnp.float32),
            pltpu.VMEM((G_PER_IMG,), jnp.float32),
            pltpu.VMEM((G_PER_IMG,), jnp.float32),
            pltpu.VMEM((_RPT,), jnp.float32),
            pltpu.VMEM((_RPT,), jnp.float32),
            pltpu.VMEM((_RPT,), jnp.float32),
        ],
    )(gid1, gt_f[:, 4], gt_f[:, 5], gt_f[:, 6])

    grid = (B // IMGS,)
    out_shapes = (
        jax.ShapeDtypeStruct((B, N, HW), jnp.float32),
        jax.ShapeDtypeStruct((B, N, HW), jnp.float32),
        jax.ShapeDtypeStruct((B, N, 2 * HW), jnp.float32),
        jax.ShapeDtypeStruct((B, N, 2 * HW), jnp.float32),
    )
    in_specs = [
        pl.BlockSpec((IMGS, N, 4), lambda i: (i, 0, 0)),
        pl.BlockSpec((IMGS, N, 1), lambda i: (i, 0, 0)),
        pl.BlockSpec((IMGS, N, 1), lambda i: (i, 0, 0)),
        pl.BlockSpec((IMGS, N, 1), lambda i: (i, 0, 0)),
        pl.BlockSpec((IMGS, N, 1), lambda i: (i, 0, 0)),
    ]
    out_specs = (
        pl.BlockSpec((IMGS, N, HW), lambda i: (i, 0, 0)),
        pl.BlockSpec((IMGS, N, HW), lambda i: (i, 0, 0)),
        pl.BlockSpec((IMGS, N, 2 * HW), lambda i: (i, 0, 0)),
        pl.BlockSpec((IMGS, N, 2 * HW), lambda i: (i, 0, 0)),
    )
    cls, clsw, reg, regw = pl.pallas_call(
        _label_kernel,
        grid=grid,
        in_specs=in_specs,
        out_specs=out_specs,
        out_shape=out_shapes,
    )(boxes, kxg.reshape(B, N, 1), kyg.reshape(B, N, 1),
      kvg.reshape(B, N, 1), flag)

    return (cls.reshape(B, N, KPS, FEAT_H, FEAT_W),
            clsw.reshape(B, N, KPS, FEAT_H, FEAT_W),
            reg.reshape(B, N, 2 * KPS, FEAT_H, FEAT_W),
            regw.reshape(B, N, 2 * KPS, FEAT_H, FEAT_W))


# speedup vs baseline: 1.6616x; 1.6616x over previous
import jax
import jax.numpy as jnp
from jax import lax
from jax.experimental import pallas as pl

FEAT_H = 16
FEAT_W = 16
HW = FEAT_H * FEAT_W
GAUSS_TH = 0.6
EXPAND = 1.0
SIGMA = 1.6
BIN_OFF = 0.5
RADIUS = 1.0
NEG_LOG_TH = 0.5108255840295616
TWO_SIG2 = 2.0 * SIGMA ** 2
IMGS = 2
N_PER_IMG = 512
G_PER_IMG = 64


def _label_kernel(boxes_ref, gt_ref, flag_ref, gid_ref, bins_ref,
                  cls_ref, clsw_ref, reg_ref, regw_ref):
    rows = IMGS * N_PER_IMG
    ng = IMGS * G_PER_IMG
    boxes = boxes_ref[...].reshape(rows, 4)
    gt = gt_ref[...].reshape(ng, 8)
    flag = flag_ref[...].reshape(rows, 1)
    gid = gid_ref[...].reshape(rows, 1)
    bin_x = bins_ref[0, 0:1]
    bin_y = bins_ref[0, 1:2]
    bxo = bins_ref[0, 2:3]
    byo = bins_ref[0, 3:4]

    goff = (lax.broadcasted_iota(jnp.int32, (rows, 1), 0)
            // N_PER_IMG) * G_PER_IMG
    gslot = gid + goff
    onehot = (gslot == lax.broadcasted_iota(jnp.int32, (rows, ng), 1)
              ).astype(jnp.float32)
    matched = jnp.dot(onehot, gt, preferred_element_type=jnp.float32,
                      precision=lax.Precision.HIGHEST)

    x1 = boxes[:, 0:1]
    y1 = boxes[:, 1:2]
    x2 = boxes[:, 2:3]
    y2 = boxes[:, 3:4]
    cx = (x1 + x2) * 0.5
    cy = (y1 + y2) * 0.5
    w = (x2 - x1 + 1.0) * EXPAND
    h = (y2 - y1 + 1.0) * EXPAND
    bx1 = cx - (w - 1.0) * 0.5
    by1 = cy - (h - 1.0) * 0.5
    bx2 = cx + (w - 1.0) * 0.5
    by2 = cy + (h - 1.0) * 0.5

    kx = matched[:, 4:5]
    ky = matched[:, 5:6]
    kv = matched[:, 6:7]

    sx = FEAT_W / (bx2 - bx1 + 1.0)
    sy = FEAT_H / (by2 - by1 + 1.0)
    x0 = (kx - bx1) * sx
    y0 = (ky - by1) * sy

    dx = bxo - x0
    dy = byo - y0
    inv2s2 = 1.0 / TWO_SIG2
    arg = dx * dx * inv2s2 + dy * dy * inv2s2
    score = jnp.exp(-arg)
    keep = arg <= NEG_LOG_TH

    vis = kv != 0.0
    pos = flag > 0
    active = pos & vis & jnp.any(keep, axis=-1, keepdims=True)
    a2 = jnp.broadcast_to(active, (rows, HW))

    cls_ref[...] = jnp.where(a2, score, -1.0).reshape(IMGS, N_PER_IMG, HW)
    clsw_ref[...] = jnp.where(a2, 1.0, 0.0).reshape(IMGS, N_PER_IMG, HW)

    m = active & keep
    off_x = (x0 - bin_x) / RADIUS
    off_y = (y0 - bin_y) / RADIUS
    zeros = jnp.zeros_like(score)
    reg = jnp.concatenate([jnp.where(m, off_x, zeros),
                           jnp.where(m, off_y, zeros)], axis=1)
    reg_ref[...] = reg.reshape(IMGS, N_PER_IMG, 2 * HW)
    rw = jnp.where(m, 1.0, 0.0)
    regw_ref[...] = jnp.concatenate([rw, rw], axis=1
                                    ).reshape(IMGS, N_PER_IMG, 2 * HW)


def kernel(boxes, gt_boxes, match_pos_flag, match_gt_id):
    B, N = boxes.shape[:2]
    KPS = 1

    flag = match_pos_flag.astype(jnp.int32).reshape(B, N, 1)
    gid = match_gt_id.astype(jnp.int32).reshape(B, N, 1)

    k = jnp.arange(HW, dtype=jnp.int32)
    bx = (k % FEAT_W).astype(jnp.float32)
    by = (k // FEAT_W).astype(jnp.float32)
    bins = jnp.stack([bx, by, bx + BIN_OFF, by + BIN_OFF]).reshape(1, 4, HW)

    grid = (B // IMGS,)
    out_shapes = (
        jax.ShapeDtypeStruct((B, N, HW), jnp.float32),
        jax.ShapeDtypeStruct((B, N, HW), jnp.float32),
        jax.ShapeDtypeStruct((B, N, 2 * HW), jnp.float32),
        jax.ShapeDtypeStruct((B, N, 2 * HW), jnp.float32),
    )
    in_specs = [
        pl.BlockSpec((IMGS, N, 4), lambda i: (i, 0, 0)),
        pl.BlockSpec((IMGS, 64, 8), lambda i: (i, 0, 0)),
        pl.BlockSpec((IMGS, N, 1), lambda i: (i, 0, 0)),
        pl.BlockSpec((IMGS, N, 1), lambda i: (i, 0, 0)),
        pl.BlockSpec((1, 4, HW), lambda i: (0, 0, 0)),
    ]
    out_specs = (
        pl.BlockSpec((IMGS, N, HW), lambda i: (i, 0, 0)),
        pl.BlockSpec((IMGS, N, HW), lambda i: (i, 0, 0)),
        pl.BlockSpec((IMGS, N, 2 * HW), lambda i: (i, 0, 0)),
        pl.BlockSpec((IMGS, N, 2 * HW), lambda i: (i, 0, 0)),
    )
    cls, clsw, reg, regw = pl.pallas_call(
        _label_kernel,
        grid=grid,
        in_specs=in_specs,
        out_specs=out_specs,
        out_shape=out_shapes,
    )(boxes, gt_boxes, flag, gid, bins)

    return (cls.reshape(B, N, KPS, FEAT_H, FEAT_W),
            clsw.reshape(B, N, KPS, FEAT_H, FEAT_W),
            reg.reshape(B, N, 2 * KPS, FEAT_H, FEAT_W),
            regw.reshape(B, N, 2 * KPS, FEAT_H, FEAT_W))


# speedup vs baseline: 1.6903x; 1.0172x over previous
import jax
import jax.numpy as jnp
from jax import lax
from jax.experimental import pallas as pl

FEAT_H = 16
FEAT_W = 16
HW = FEAT_H * FEAT_W
GAUSS_TH = 0.6
EXPAND = 1.0
SIGMA = 1.6
BIN_OFF = 0.5
RADIUS = 1.0
NEG_LOG_TH = 0.5108255840295616
TWO_SIG2 = 2.0 * SIGMA ** 2
IMGS = 2
N_PER_IMG = 512
G_PER_IMG = 64


def _label_kernel(boxes_ref, gt_ref, flag_ref, gid_ref,
                  cls_ref, clsw_ref, reg_ref, regw_ref):
    rows = IMGS * N_PER_IMG
    ng = IMGS * G_PER_IMG
    boxes = boxes_ref[...].reshape(rows, 4)
    gt = gt_ref[...].reshape(ng, 8)
    flag = flag_ref[...].reshape(rows, 1)
    gid = gid_ref[...].reshape(rows, 1)

    goff = (lax.broadcasted_iota(jnp.int32, (rows, 1), 0)
            // N_PER_IMG) * G_PER_IMG
    gslot = gid + goff
    onehot = (gslot == lax.broadcasted_iota(jnp.int32, (rows, ng), 1)
              ).astype(jnp.float32)
    matched = jnp.dot(onehot, gt, preferred_element_type=jnp.float32,
                      precision=lax.Precision.HIGHEST)

    x1 = boxes[:, 0:1]
    y1 = boxes[:, 1:2]
    x2 = boxes[:, 2:3]
    y2 = boxes[:, 3:4]
    cx = (x1 + x2) * 0.5
    cy = (y1 + y2) * 0.5
    w = (x2 - x1 + 1.0) * EXPAND
    h = (y2 - y1 + 1.0) * EXPAND
    bx1 = cx - (w - 1.0) * 0.5
    by1 = cy - (h - 1.0) * 0.5
    bx2 = cx + (w - 1.0) * 0.5
    by2 = cy + (h - 1.0) * 0.5

    kx = matched[:, 4:5]
    ky = matched[:, 5:6]
    kv = matched[:, 6:7]

    sx = FEAT_W / (bx2 - bx1 + 1.0)
    sy = FEAT_H / (by2 - by1 + 1.0)
    x0 = (kx - bx1) * sx
    y0 = (ky - by1) * sy

    col = lax.broadcasted_iota(jnp.int32, (rows, HW), 1)
    bin_x = (col % FEAT_W).astype(jnp.float32)
    bin_y = (col // FEAT_W).astype(jnp.float32)

    dx = bin_x + BIN_OFF - x0
    dy = bin_y + BIN_OFF - y0
    inv2s2 = 1.0 / TWO_SIG2
    arg = dx * dx * inv2s2 + dy * dy * inv2s2
    score = jnp.exp(-arg)
    keep = arg <= NEG_LOG_TH

    vis = kv != 0.0
    pos = flag > 0
    active = pos & vis & jnp.any(keep, axis=-1, keepdims=True)

    cls_ref[...] = jnp.where(active, score, -1.0).reshape(IMGS, N_PER_IMG, HW)
    clsw_ref[...] = (jnp.where(active, 1.0, 0.0) * jnp.ones_like(score)
                     ).reshape(IMGS, N_PER_IMG, HW)

    m = active & keep
    off_x = (x0 - bin_x) / RADIUS
    off_y = (y0 - bin_y) / RADIUS
    zeros = jnp.zeros_like(score)
    reg = jnp.concatenate([jnp.where(m, off_x, zeros),
                           jnp.where(m, off_y, zeros)], axis=1)
    reg_ref[...] = reg.reshape(IMGS, N_PER_IMG, 2 * HW)
    rw = jnp.where(m, 1.0, 0.0)
    regw_ref[...] = jnp.concatenate([rw, rw], axis=1
                                    ).reshape(IMGS, N_PER_IMG, 2 * HW)


def kernel(boxes, gt_boxes, match_pos_flag, match_gt_id):
    B, N = boxes.shape[:2]
    KPS = 1

    flag = match_pos_flag.astype(jnp.int32).reshape(B, N, 1)
    gid = match_gt_id.astype(jnp.int32).reshape(B, N, 1)

    grid = (B // IMGS,)
    out_shapes = (
        jax.ShapeDtypeStruct((B, N, HW), jnp.float32),
        jax.ShapeDtypeStruct((B, N, HW), jnp.float32),
        jax.ShapeDtypeStruct((B, N, 2 * HW), jnp.float32),
        jax.ShapeDtypeStruct((B, N, 2 * HW), jnp.float32),
    )
    in_specs = [
        pl.BlockSpec((IMGS, N, 4), lambda i: (i, 0, 0)),
        pl.BlockSpec((IMGS, 64, 8), lambda i: (i, 0, 0)),
        pl.BlockSpec((IMGS, N, 1), lambda i: (i, 0, 0)),
        pl.BlockSpec((IMGS, N, 1), lambda i: (i, 0, 0)),
    ]
    out_specs = (
        pl.BlockSpec((IMGS, N, HW), lambda i: (i, 0, 0)),
        pl.BlockSpec((IMGS, N, HW), lambda i: (i, 0, 0)),
        pl.BlockSpec((IMGS, N, 2 * HW), lambda i: (i, 0, 0)),
        pl.BlockSpec((IMGS, N, 2 * HW), lambda i: (i, 0, 0)),
    )
    cls, clsw, reg, regw = pl.pallas_call(
        _label_kernel,
        grid=grid,
        in_specs=in_specs,
        out_specs=out_specs,
        out_shape=out_shapes,
    )(boxes, gt_boxes, flag, gid)

    return (cls.reshape(B, N, KPS, FEAT_H, FEAT_W),
            clsw.reshape(B, N, KPS, FEAT_H, FEAT_W),
            reg.reshape(B, N, 2 * KPS, FEAT_H, FEAT_W),
            regw.reshape(B, N, 2 * KPS, FEAT_H, FEAT_W))
